# baseline (speedup 1.0000x reference)
import functools

import jax
import jax.numpy as jnp
from jax import lax
from jax.experimental import pallas as pl
from jax.experimental.pallas import tpu as pltpu
from jax.experimental.pallas import tpu_sc as plsc

NC = 2
NS = 16
NW = NC * NS
K = 128
BLK0 = 128
BLK1 = 32
CH = 8

_vector_mesh = plsc.VectorSubcoreMesh(core_axis_name="c", subcore_axis_name="s")


def _deg_body(blk, rpt, dst_hbm, deg_out, idx_v, ones_v, z_v, deg_sh):
    c = lax.axis_index("c")
    s = lax.axis_index("s")
    slab = c * NS + s
    z16 = jnp.zeros((16,), jnp.float32)
    o16 = jnp.ones((16,), jnp.float32)
    for i in range(K // 16):
        ones_v[pl.ds(i * 16, 16)] = o16
    for i in range(rpt // 16):
        z_v[pl.ds(i * 16, 16)] = z16
    pltpu.sync_copy(z_v, deg_sh.at[pl.ds(s * rpt, rpt)])
    pltpu.sync_copy(dst_hbm.at[slab], idx_v)
    plsc.subcore_barrier()

    @pl.loop(0, blk)
    def _(j):
        pltpu.sync_copy(ones_v, deg_sh.at[idx_v.at[j]], add=True)

    plsc.subcore_barrier()
    pltpu.sync_copy(deg_sh.at[pl.ds(s * rpt, rpt)],
                    deg_out.at[c, pl.ds(s * rpt, rpt)])


def _agg_body(rpt, d, hw_hbm, sd_hbm, out_hbm,
              sda_v, sdb_v, bufa, bufb, z_v, acc_sh,
              sca, scb, sga, sgb):
    c = lax.axis_index("c")
    s = lax.axis_index("s")
    z16 = jnp.zeros((16,), jnp.float32)
    for r in range(16):
        for i in range(d // 16):
            z_v[r, pl.ds(i * 16, 16)] = z16

    @pl.loop(0, rpt // 16)
    def _(i):
        pltpu.sync_copy(z_v, acc_sh.at[pl.ds(s * rpt + i * 16, 16)])

    base = c * BLK0
    cnt = jnp.where(c == 0, BLK0, BLK1)
    nch = cnt // CH
    npair = nch // 2
    pltpu.sync_copy(sd_hbm.at[s, pl.ds(base, CH)], sda_v)
    plsc.subcore_barrier()

    pltpu.async_copy(hw_hbm.at[sda_v.at[0, 0]], bufa, sga)

    @pl.loop(0, npair)
    def _(g):
        for icur, inxt, isn, hoff in (
                (sda_v, sdb_v, scb, 0), (sdb_v, sda_v, sca, 1)):
            ch = g * 2 + hoff
            not_last = ch < nch - 1

            @pl.when(not_last)
            def _():
                pltpu.async_copy(
                    sd_hbm.at[s, pl.ds(base + (ch + 1) * CH, CH)], inxt, isn)

            for r in range(CH):
                buf, sem = (bufa, sga) if r % 2 == 0 else (bufb, sgb)
                nbuf, nsem = (bufb, sgb) if r % 2 == 0 else (bufa, sga)
                if r < CH - 1:
                    pltpu.async_copy(hw_hbm.at[icur.at[r + 1, 0]], nbuf, nsem)
                else:
                    @pl.when(not_last)
                    def _():
                        pltpu.make_async_copy(
                            sd_hbm.at[s, pl.ds(base + (ch + 1) * CH, CH)],
                            inxt, isn).wait()
                        pltpu.async_copy(hw_hbm.at[inxt.at[0, 0]], nbuf, nsem)
                pltpu.make_async_copy(hw_hbm.at[icur.at[r, 0]], buf, sem).wait()
                pltpu.sync_copy(buf, acc_sh.at[icur.at[r, 1]], add=True)

    plsc.subcore_barrier()
    pltpu.sync_copy(acc_sh.at[pl.ds(s * rpt, rpt)],
                    out_hbm.at[c, pl.ds(s * rpt, rpt)])


def _deg_kernel(n_pad, blk):
    rpt = n_pad // NS
    return pl.kernel(
        functools.partial(_deg_body, blk, rpt),
        out_type=jax.ShapeDtypeStruct((NC, n_pad), jnp.float32),
        mesh=_vector_mesh,
        scratch_types=[
            pltpu.VMEM((blk, K), jnp.int32),
            pltpu.VMEM((K,), jnp.float32),
            pltpu.VMEM((rpt,), jnp.float32),
            pltpu.VMEM_SHARED((n_pad,), jnp.float32),
        ],
    )


def _agg_kernel(n_pad, d):
    rpt = n_pad // NS
    return pl.kernel(
        functools.partial(_agg_body, rpt, d),
        out_type=jax.ShapeDtypeStruct((NC, n_pad, d), jnp.float32),
        mesh=_vector_mesh,
        scratch_types=[
            pltpu.VMEM((CH, 2, K), jnp.int32),
            pltpu.VMEM((CH, 2, K), jnp.int32),
            pltpu.VMEM((K, d), jnp.float32),
            pltpu.VMEM((K, d), jnp.float32),
            pltpu.VMEM((16, d), jnp.float32),
            pltpu.VMEM_SHARED((n_pad, d), jnp.float32),
            pltpu.SemaphoreType.DMA,
            pltpu.SemaphoreType.DMA,
            pltpu.SemaphoreType.DMA,
            pltpu.SemaphoreType.DMA,
        ],
    )


def _mm_body(x_ref, w_ref, o_ref):
    o_ref[...] = jnp.dot(x_ref[...], w_ref[...],
                         preferred_element_type=jnp.float32)


def _scale_body(degt_ref, xw_ref, o_ref):
    dinv = lax.rsqrt(degt_ref[...].sum(axis=1, keepdims=True) + 1.0)
    o_ref[...] = xw_ref[...] * dinv


def _mid_body(degt_ref, p0_ref, p1_ref, hws_ref, b_ref, w_ref, o_ref):
    dinv = lax.rsqrt(degt_ref[...].sum(axis=1, keepdims=True) + 1.0)
    h = (p0_ref[...] + p1_ref[...] + hws_ref[...]) * dinv + b_ref[...]
    h = jnp.maximum(h, 0.0)
    o_ref[...] = jnp.dot(h, w_ref[...],
                         preferred_element_type=jnp.float32) * dinv


def _final_body(degt_ref, q0_ref, q1_ref, hws_ref, b_ref, o_ref):
    dinv = lax.rsqrt(degt_ref[...].sum(axis=1, keepdims=True) + 1.0)
    o_ref[...] = (q0_ref[...] + q1_ref[...] + hws_ref[...]) * dinv + b_ref[...]


def _asym_slabs(v, e0):
    r0 = v[:e0].reshape(NS, BLK0, K)
    r1 = v[e0:].reshape(NS, BLK1, K)
    return jnp.concatenate([r0, r1], axis=1)


def kernel(x, edge_index, W1, b1, W2, b2):
    n, d = x.shape
    e = edge_index.shape[1]

    n_pad = ((n + NS * 16) // (NS * 16)) * (NS * 16)
    src = edge_index[0].astype(jnp.int32)
    dst = edge_index[1].astype(jnp.int32)

    blk_d = -(-e // (NW * K))
    pad_d = NW * blk_d * K - e
    dst3_d = jnp.concatenate(
        [dst, jnp.full((pad_d,), n, jnp.int32)]).reshape(NW, blk_d, K)

    e_pad = NS * (BLK0 + BLK1) * K
    e0 = NS * BLK0 * K
    srcp = jnp.concatenate([src, jnp.zeros((e_pad - e,), jnp.int32)])
    dstp = jnp.concatenate([dst, jnp.full((e_pad - e,), n, jnp.int32)])
    sd3 = jnp.stack([_asym_slabs(srcp, e0), _asym_slabs(dstp, e0)], axis=2)

    f32 = jnp.float32
    x_pad = jnp.pad(x, ((0, n_pad - n), (0, 0)))

    deg_parts = _deg_kernel(n_pad, blk_d)(dst3_d)
    xw1 = pl.pallas_call(
        _mm_body, out_shape=jax.ShapeDtypeStruct((n_pad, d), f32))(x_pad, W1)
    degt = deg_parts.T

    hws1 = pl.pallas_call(
        _scale_body, out_shape=jax.ShapeDtypeStruct((n_pad, d), f32))(degt, xw1)

    agg = _agg_kernel(n_pad, d)
    p = agg(hws1, sd3)
    hws2 = pl.pallas_call(
        _mid_body, out_shape=jax.ShapeDtypeStruct((n_pad, d), f32))(
            degt, p[0], p[1], hws1, b1.reshape(1, d), W2)

    q = agg(hws2, sd3)
    out = pl.pallas_call(
        _final_body, out_shape=jax.ShapeDtypeStruct((n_pad, d), f32))(
            degt, q[0], q[1], hws2, b2.reshape(1, d))
    return out[:n]


# speedup vs baseline: 1.1419x; 1.1419x over previous
import functools

import jax
import jax.numpy as jnp
from jax import lax
from jax.experimental import pallas as pl
from jax.experimental.pallas import tpu as pltpu
from jax.experimental.pallas import tpu_sc as plsc

NC = 2
NS = 16
NW = NC * NS
K = 128
BLK0 = 144
BLK1 = 16
CH = 8

_vector_mesh = plsc.VectorSubcoreMesh(core_axis_name="c", subcore_axis_name="s")


def _deg_body(blk, rpt, dst_hbm, deg_out, idx_v, ones_v, z_v, deg_sh):
    c = lax.axis_index("c")
    s = lax.axis_index("s")
    slab = c * NS + s
    z16 = jnp.zeros((16,), jnp.float32)
    o16 = jnp.ones((16,), jnp.float32)
    for i in range(K // 16):
        ones_v[pl.ds(i * 16, 16)] = o16
    for i in range(rpt // 16):
        z_v[pl.ds(i * 16, 16)] = z16
    pltpu.sync_copy(z_v, deg_sh.at[pl.ds(s * rpt, rpt)])
    pltpu.sync_copy(dst_hbm.at[slab], idx_v)
    plsc.subcore_barrier()

    @pl.loop(0, blk)
    def _(j):
        pltpu.sync_copy(ones_v, deg_sh.at[idx_v.at[j]], add=True)

    plsc.subcore_barrier()
    pltpu.sync_copy(deg_sh.at[pl.ds(s * rpt, rpt)],
                    deg_out.at[c, pl.ds(s * rpt, rpt)])


def _agg_body(rpt, d, hw_hbm, sd_hbm, out_hbm,
              sda_v, sdb_v, bufa, bufb, z_v, acc_sh,
              sca, scb, sga, sgb):
    c = lax.axis_index("c")
    s = lax.axis_index("s")
    z16 = jnp.zeros((16,), jnp.float32)
    for r in range(16):
        for i in range(d // 16):
            z_v[r, pl.ds(i * 16, 16)] = z16

    @pl.loop(0, rpt // 16)
    def _(i):
        pltpu.sync_copy(z_v, acc_sh.at[pl.ds(s * rpt + i * 16, 16)])

    base = c * BLK0
    cnt = jnp.where(c == 0, BLK0, BLK1)
    nch = cnt // CH
    npair = nch // 2
    pltpu.sync_copy(sd_hbm.at[s, pl.ds(base, CH)], sda_v)
    plsc.subcore_barrier()

    pltpu.async_copy(hw_hbm.at[sda_v.at[0, 0]], bufa, sga)

    @pl.loop(0, npair)
    def _(g):
        for icur, inxt, isn, hoff in (
                (sda_v, sdb_v, scb, 0), (sdb_v, sda_v, sca, 1)):
            ch = g * 2 + hoff
            not_last = ch < nch - 1

            @pl.when(not_last)
            def _():
                pltpu.async_copy(
                    sd_hbm.at[s, pl.ds(base + (ch + 1) * CH, CH)], inxt, isn)

            for r in range(CH):
                buf, sem = (bufa, sga) if r % 2 == 0 else (bufb, sgb)
                nbuf, nsem = (bufb, sgb) if r % 2 == 0 else (bufa, sga)
                if r < CH - 1:
                    pltpu.async_copy(hw_hbm.at[icur.at[r + 1, 0]], nbuf, nsem)
                else:
                    @pl.when(not_last)
                    def _():
                        pltpu.make_async_copy(
                            sd_hbm.at[s, pl.ds(base + (ch + 1) * CH, CH)],
                            inxt, isn).wait()
                        pltpu.async_copy(hw_hbm.at[inxt.at[0, 0]], nbuf, nsem)
                pltpu.make_async_copy(hw_hbm.at[icur.at[r, 0]], buf, sem).wait()
                pltpu.sync_copy(buf, acc_sh.at[icur.at[r, 1]], add=True)

    plsc.subcore_barrier()
    pltpu.sync_copy(acc_sh.at[pl.ds(s * rpt, rpt)],
                    out_hbm.at[c, pl.ds(s * rpt, rpt)])


def _deg_kernel(n_pad, blk):
    rpt = n_pad // NS
    return pl.kernel(
        functools.partial(_deg_body, blk, rpt),
        out_type=jax.ShapeDtypeStruct((NC, n_pad), jnp.float32),
        mesh=_vector_mesh,
        scratch_types=[
            pltpu.VMEM((blk, K), jnp.int32),
            pltpu.VMEM((K,), jnp.float32),
            pltpu.VMEM((rpt,), jnp.float32),
            pltpu.VMEM_SHARED((n_pad,), jnp.float32),
        ],
    )


def _agg_kernel(n_pad, d):
    rpt = n_pad // NS
    return pl.kernel(
        functools.partial(_agg_body, rpt, d),
        out_type=jax.ShapeDtypeStruct((NC, n_pad, d), jnp.float32),
        mesh=_vector_mesh,
        scratch_types=[
            pltpu.VMEM((CH, 2, K), jnp.int32),
            pltpu.VMEM((CH, 2, K), jnp.int32),
            pltpu.VMEM((K, d), jnp.float32),
            pltpu.VMEM((K, d), jnp.float32),
            pltpu.VMEM((16, d), jnp.float32),
            pltpu.VMEM_SHARED((n_pad, d), jnp.float32),
            pltpu.SemaphoreType.DMA,
            pltpu.SemaphoreType.DMA,
            pltpu.SemaphoreType.DMA,
            pltpu.SemaphoreType.DMA,
        ],
    )


def _mm_body(x_ref, w_ref, o_ref):
    o_ref[...] = jnp.dot(x_ref[...], w_ref[...],
                         preferred_element_type=jnp.float32)


def _scale_body(degt_ref, xw_ref, o_ref):
    dinv = lax.rsqrt(degt_ref[...].sum(axis=1, keepdims=True) + 1.0)
    o_ref[...] = xw_ref[...] * dinv


def _mid_body(degt_ref, p0_ref, p1_ref, hws_ref, b_ref, w_ref, o_ref):
    dinv = lax.rsqrt(degt_ref[...].sum(axis=1, keepdims=True) + 1.0)
    h = (p0_ref[...] + p1_ref[...] + hws_ref[...]) * dinv + b_ref[...]
    h = jnp.maximum(h, 0.0)
    o_ref[...] = jnp.dot(h, w_ref[...],
                         preferred_element_type=jnp.float32) * dinv


def _final_body(degt_ref, q0_ref, q1_ref, hws_ref, b_ref, o_ref):
    dinv = lax.rsqrt(degt_ref[...].sum(axis=1, keepdims=True) + 1.0)
    o_ref[...] = (q0_ref[...] + q1_ref[...] + hws_ref[...]) * dinv + b_ref[...]


def _asym_slabs(v, e0):
    r0 = v[:e0].reshape(NS, BLK0, K)
    r1 = v[e0:].reshape(NS, BLK1, K)
    return jnp.concatenate([r0, r1], axis=1)


def kernel(x, edge_index, W1, b1, W2, b2):
    n, d = x.shape
    e = edge_index.shape[1]

    n_pad = ((n + NS * 16) // (NS * 16)) * (NS * 16)
    src = edge_index[0].astype(jnp.int32)
    dst = edge_index[1].astype(jnp.int32)

    blk_d = -(-e // (NW * K))
    pad_d = NW * blk_d * K - e
    dst3_d = jnp.concatenate(
        [dst, jnp.full((pad_d,), n, jnp.int32)]).reshape(NW, blk_d, K)

    e_pad = NS * (BLK0 + BLK1) * K
    e0 = NS * BLK0 * K
    srcp = jnp.concatenate([src, jnp.zeros((e_pad - e,), jnp.int32)])
    dstp = jnp.concatenate([dst, jnp.full((e_pad - e,), n, jnp.int32)])
    sd3 = jnp.stack([_asym_slabs(srcp, e0), _asym_slabs(dstp, e0)], axis=2)

    f32 = jnp.float32
    x_pad = jnp.pad(x, ((0, n_pad - n), (0, 0)))

    deg_parts = _deg_kernel(n_pad, blk_d)(dst3_d)
    xw1 = pl.pallas_call(
        _mm_body, out_shape=jax.ShapeDtypeStruct((n_pad, d), f32))(x_pad, W1)
    degt = deg_parts.T

    hws1 = pl.pallas_call(
        _scale_body, out_shape=jax.ShapeDtypeStruct((n_pad, d), f32))(degt, xw1)

    agg = _agg_kernel(n_pad, d)
    p = agg(hws1, sd3)
    hws2 = pl.pallas_call(
        _mid_body, out_shape=jax.ShapeDtypeStruct((n_pad, d), f32))(
            degt, p[0], p[1], hws1, b1.reshape(1, d), W2)

    q = agg(hws2, sd3)
    out = pl.pallas_call(
        _final_body, out_shape=jax.ShapeDtypeStruct((n_pad, d), f32))(
            degt, q[0], q[1], hws2, b2.reshape(1, d))
    return out[:n]


# speedup vs baseline: 1.1474x; 1.0048x over previous
import functools

import jax
import jax.numpy as jnp
from jax import lax
from jax.experimental import pallas as pl
from jax.experimental.pallas import tpu as pltpu
from jax.experimental.pallas import tpu_sc as plsc

NC = 2
NS = 16
NW = NC * NS
K = 128
BLK0 = 144
BLK1 = 16
CH = 8

_vector_mesh = plsc.VectorSubcoreMesh(core_axis_name="c", subcore_axis_name="s")


def _deg_body(blk, rpt, dst_hbm, deg_out, idx_v, ones_v, z_v, deg_sh):
    c = lax.axis_index("c")
    s = lax.axis_index("s")
    slab = c * NS + s
    z16 = jnp.zeros((16,), jnp.float32)
    o16 = jnp.ones((16,), jnp.float32)
    for i in range(K // 16):
        ones_v[pl.ds(i * 16, 16)] = o16
    for i in range(rpt // 16):
        z_v[pl.ds(i * 16, 16)] = z16
    pltpu.sync_copy(z_v, deg_sh.at[pl.ds(s * rpt, rpt)])
    pltpu.sync_copy(dst_hbm.at[slab], idx_v)
    plsc.subcore_barrier()

    @pl.loop(0, blk)
    def _(j):
        pltpu.sync_copy(ones_v, deg_sh.at[idx_v.at[j]], add=True)

    plsc.subcore_barrier()
    pltpu.sync_copy(deg_sh.at[pl.ds(s * rpt, rpt)],
                    deg_out.at[c, pl.ds(s * rpt, rpt)])


def _agg_body(rpt, d, hw_hbm, sd_hbm, out_hbm,
              sda_v, sdb_v, bufa, bufb, z_v, acc_sh,
              sca, scb, sga, sgb):
    NCH0 = BLK0 // CH
    NP0 = NCH0 // 2
    c = lax.axis_index("c")
    s = lax.axis_index("s")
    z16 = jnp.zeros((16,), jnp.float32)
    for r in range(16):
        for i in range(d // 16):
            z_v[r, pl.ds(i * 16, 16)] = z16

    @pl.loop(0, rpt // 16)
    def _(i):
        pltpu.sync_copy(z_v, acc_sh.at[pl.ds(s * rpt + i * 16, 16)])

    plsc.subcore_barrier()

    @pl.when(c == 0)
    def _():
        pltpu.sync_copy(sd_hbm.at[s, pl.ds(0, CH)], sda_v)
        pltpu.async_copy(hw_hbm.at[sda_v.at[0, 0]], bufa, sga)

        @pl.loop(0, NP0)
        def _(g):
            for icur, inxt, isn, hoff in (
                    (sda_v, sdb_v, scb, 0), (sdb_v, sda_v, sca, 1)):
                ch = g * 2 + hoff
                not_last = (g < NP0 - 1) if hoff == 1 else True

                def _prefetch():
                    pltpu.async_copy(
                        sd_hbm.at[s, pl.ds((ch + 1) * CH, CH)], inxt, isn)

                if hoff == 0:
                    _prefetch()
                else:
                    pl.when(not_last)(_prefetch)

                for r in range(CH):
                    buf, sem = (bufa, sga) if r % 2 == 0 else (bufb, sgb)
                    nbuf, nsem = (bufb, sgb) if r % 2 == 0 else (bufa, sga)
                    if r < CH - 1:
                        pltpu.async_copy(hw_hbm.at[icur.at[r + 1, 0]],
                                         nbuf, nsem)
                    else:
                        def _tail():
                            pltpu.make_async_copy(
                                sd_hbm.at[s, pl.ds((ch + 1) * CH, CH)],
                                inxt, isn).wait()
                            pltpu.async_copy(hw_hbm.at[inxt.at[0, 0]],
                                             nbuf, nsem)
                        if hoff == 0:
                            _tail()
                        else:
                            pl.when(not_last)(_tail)
                    pltpu.make_async_copy(hw_hbm.at[icur.at[r, 0]],
                                          buf, sem).wait()
                    pltpu.sync_copy(buf, acc_sh.at[icur.at[r, 1]], add=True)

    @pl.when(c == 1)
    def _():
        @pl.loop(0, BLK1 // CH)
        def _(ch):
            pltpu.sync_copy(sd_hbm.at[s, pl.ds(BLK0 + ch * CH, CH)], sda_v)
            for r in range(CH):
                pltpu.sync_copy(hw_hbm.at[sda_v.at[r, 0]], bufa)
                pltpu.sync_copy(bufa, acc_sh.at[sda_v.at[r, 1]], add=True)

    plsc.subcore_barrier()
    pltpu.sync_copy(acc_sh.at[pl.ds(s * rpt, rpt)],
                    out_hbm.at[c, pl.ds(s * rpt, rpt)])


def _deg_kernel(n_pad, blk):
    rpt = n_pad // NS
    return pl.kernel(
        functools.partial(_deg_body, blk, rpt),
        out_type=jax.ShapeDtypeStruct((NC, n_pad), jnp.float32),
        mesh=_vector_mesh,
        scratch_types=[
            pltpu.VMEM((blk, K), jnp.int32),
            pltpu.VMEM((K,), jnp.float32),
            pltpu.VMEM((rpt,), jnp.float32),
            pltpu.VMEM_SHARED((n_pad,), jnp.float32),
        ],
    )


def _agg_kernel(n_pad, d):
    rpt = n_pad // NS
    return pl.kernel(
        functools.partial(_agg_body, rpt, d),
        out_type=jax.ShapeDtypeStruct((NC, n_pad, d), jnp.float32),
        mesh=_vector_mesh,
        scratch_types=[
            pltpu.VMEM((CH, 2, K), jnp.int32),
            pltpu.VMEM((CH, 2, K), jnp.int32),
            pltpu.VMEM((K, d), jnp.float32),
            pltpu.VMEM((K, d), jnp.float32),
            pltpu.VMEM((16, d), jnp.float32),
            pltpu.VMEM_SHARED((n_pad, d), jnp.float32),
            pltpu.SemaphoreType.DMA,
            pltpu.SemaphoreType.DMA,
            pltpu.SemaphoreType.DMA,
            pltpu.SemaphoreType.DMA,
        ],
    )


def _mm_body(x_ref, w_ref, o_ref):
    o_ref[...] = jnp.dot(x_ref[...], w_ref[...],
                         preferred_element_type=jnp.float32)


def _scale_body(degt_ref, xw_ref, o_ref):
    dinv = lax.rsqrt(degt_ref[...].sum(axis=1, keepdims=True) + 1.0)
    o_ref[...] = xw_ref[...] * dinv


def _mid_body(degt_ref, p0_ref, p1_ref, hws_ref, b_ref, w_ref, o_ref):
    dinv = lax.rsqrt(degt_ref[...].sum(axis=1, keepdims=True) + 1.0)
    h = (p0_ref[...] + p1_ref[...] + hws_ref[...]) * dinv + b_ref[...]
    h = jnp.maximum(h, 0.0)
    o_ref[...] = jnp.dot(h, w_ref[...],
                         preferred_element_type=jnp.float32) * dinv


def _final_body(degt_ref, q0_ref, q1_ref, hws_ref, b_ref, o_ref):
    dinv = lax.rsqrt(degt_ref[...].sum(axis=1, keepdims=True) + 1.0)
    o_ref[...] = (q0_ref[...] + q1_ref[...] + hws_ref[...]) * dinv + b_ref[...]


def _asym_slabs(v, e0):
    r0 = v[:e0].reshape(NS, BLK0, K)
    r1 = v[e0:].reshape(NS, BLK1, K)
    return jnp.concatenate([r0, r1], axis=1)


def kernel(x, edge_index, W1, b1, W2, b2):
    n, d = x.shape
    e = edge_index.shape[1]

    n_pad = ((n + NS * 16) // (NS * 16)) * (NS * 16)
    src = edge_index[0].astype(jnp.int32)
    dst = edge_index[1].astype(jnp.int32)

    blk_d = -(-e // (NW * K))
    pad_d = NW * blk_d * K - e
    dst3_d = jnp.concatenate(
        [dst, jnp.full((pad_d,), n, jnp.int32)]).reshape(NW, blk_d, K)

    e_pad = NS * (BLK0 + BLK1) * K
    e0 = NS * BLK0 * K
    srcp = jnp.concatenate([src, jnp.zeros((e_pad - e,), jnp.int32)])
    dstp = jnp.concatenate([dst, jnp.full((e_pad - e,), n, jnp.int32)])
    sd3 = jnp.stack([_asym_slabs(srcp, e0), _asym_slabs(dstp, e0)], axis=2)

    f32 = jnp.float32
    x_pad = jnp.pad(x, ((0, n_pad - n), (0, 0)))

    deg_parts = _deg_kernel(n_pad, blk_d)(dst3_d)
    xw1 = pl.pallas_call(
        _mm_body, out_shape=jax.ShapeDtypeStruct((n_pad, d), f32))(x_pad, W1)
    degt = deg_parts.T

    hws1 = pl.pallas_call(
        _scale_body, out_shape=jax.ShapeDtypeStruct((n_pad, d), f32))(degt, xw1)

    agg = _agg_kernel(n_pad, d)
    p = agg(hws1, sd3)
    hws2 = pl.pallas_call(
        _mid_body, out_shape=jax.ShapeDtypeStruct((n_pad, d), f32))(
            degt, p[0], p[1], hws1, b1.reshape(1, d), W2)

    q = agg(hws2, sd3)
    out = pl.pallas_call(
        _final_body, out_shape=jax.ShapeDtypeStruct((n_pad, d), f32))(
            degt, q[0], q[1], hws2, b2.reshape(1, d))
    return out[:n]
